# Initial kernel scaffold; baseline (speedup 1.0000x reference)
import jax
import jax.numpy as jnp
from jax.experimental import pallas as pl


def kernel(x, table):
    raise NotImplementedError("write your pallas kernel here")


# speedup vs baseline: 2.8665x; 2.8665x over previous
import functools

import jax
import jax.numpy as jnp
from jax import lax
from jax.experimental import pallas as pl
from jax.experimental.pallas import tpu as pltpu
from jax.experimental.pallas import tpu_sc as plsc

CHUNK = 128


DP = 128


def _make_gather(N, V, D, num_cores, num_subcores):
    NW = num_cores * num_subcores
    per_w = N // NW
    n_chunks = per_w // CHUNK
    mesh = plsc.VectorSubcoreMesh(core_axis_name="c", subcore_axis_name="s")

    @functools.partial(
        pl.kernel,
        out_type=jax.ShapeDtypeStruct((N, DP), jnp.float32),
        mesh=mesh,
        scratch_types=[
            pltpu.VMEM((per_w,), jnp.int32),
            pltpu.VMEM((CHUNK, DP), jnp.float32),
            pltpu.SemaphoreType.DMA,
        ],
    )
    def gather_kernel(idx_hbm, table_hbm, out_hbm, idx_v, rows_v, sem):
        wid = lax.axis_index("s") * num_cores + lax.axis_index("c")
        base = wid * per_w
        pltpu.sync_copy(idx_hbm.at[pl.ds(base, per_w)], idx_v)

        def body(j, carry):
            off = pl.multiple_of(j * CHUNK, CHUNK)
            pltpu.async_copy(
                table_hbm.at[idx_v.at[pl.ds(off, CHUNK)]], rows_v, sem
            ).wait()
            pltpu.sync_copy(rows_v, out_hbm.at[pl.ds(base + off, CHUNK)])
            return carry

        lax.fori_loop(0, n_chunks, body, 0)

    return gather_kernel


def kernel(x, table):
    B, L = x.shape
    V, D = table.shape
    N = B * L
    info = plsc.get_sparse_core_info()
    gather = _make_gather(N, V, D, info.num_cores, info.num_subcores)
    table_p = jnp.pad(table, ((0, 0), (0, DP - D)))
    out = gather(x.reshape(N).astype(jnp.int32), table_p)
    return out[:, :D].reshape(B, L, D)


# speedup vs baseline: 3.2193x; 1.1231x over previous
import functools

import jax
import jax.numpy as jnp
from jax import lax
from jax.experimental import pallas as pl
from jax.experimental.pallas import tpu as pltpu
from jax.experimental.pallas import tpu_sc as plsc

CHUNK = 128
NBUF = 4
DP = 128


def _make_gather(N, V, D, num_cores, num_subcores):
    NW = num_cores * num_subcores
    per_w = N // NW
    n_chunks = per_w // CHUNK
    assert n_chunks % NBUF == 0
    n_groups = n_chunks // NBUF
    mesh = plsc.VectorSubcoreMesh(core_axis_name="c", subcore_axis_name="s")

    @functools.partial(
        pl.kernel,
        out_type=jax.ShapeDtypeStruct((N, DP), jnp.float32),
        mesh=mesh,
        scratch_types=[
            pltpu.VMEM((per_w,), jnp.int32),
            pltpu.VMEM((NBUF, CHUNK, DP), jnp.float32),
            pltpu.SemaphoreType.DMA((NBUF,)),
            pltpu.SemaphoreType.DMA((NBUF,)),
        ],
    )
    def gather_kernel(idx_hbm, table_hbm, out_hbm, idx_v, rows_v, g_sem, w_sem):
        wid = lax.axis_index("s") * num_cores + lax.axis_index("c")
        base = wid * per_w
        pltpu.sync_copy(idx_hbm.at[pl.ds(base, per_w)], idx_v)

        def start_gather(j, b):
            off = pl.multiple_of(j * CHUNK, CHUNK)
            pltpu.make_async_copy(
                table_hbm.at[idx_v.at[pl.ds(off, CHUNK)]],
                rows_v.at[b],
                g_sem.at[b],
            ).start()

        def wait_gather(b):
            pltpu.make_async_copy(
                table_hbm.at[idx_v.at[pl.ds(0, CHUNK)]],
                rows_v.at[b],
                g_sem.at[b],
            ).wait()

        def start_write(j, b):
            off = pl.multiple_of(j * CHUNK, CHUNK)
            pltpu.make_async_copy(
                rows_v.at[b],
                out_hbm.at[pl.ds(base + off, CHUNK)],
                w_sem.at[b],
            ).start()

        def wait_write(b):
            pltpu.make_async_copy(
                rows_v.at[b],
                out_hbm.at[pl.ds(base, CHUNK)],
                w_sem.at[b],
            ).wait()

        for b in range(NBUF):
            start_gather(b, b)

        def body(g, carry):
            for b in range(NBUF):
                j = g * NBUF + b
                wait_gather(b)
                start_write(j, b)

                @pl.when(g < n_groups - 1)
                def _():
                    wait_write(b)
                    start_gather(j + NBUF, b)

            return carry

        lax.fori_loop(0, n_groups, body, 0)
        for b in range(NBUF):
            wait_write(b)

    return gather_kernel


def kernel(x, table):
    B, L = x.shape
    V, D = table.shape
    N = B * L
    info = plsc.get_sparse_core_info()
    gather = _make_gather(N, V, D, info.num_cores, info.num_subcores)
    table_p = jnp.pad(table, ((0, 0), (0, DP - D)))
    out = gather(x.reshape(N).astype(jnp.int32), table_p)
    return out[:, :D].reshape(B, L, D)


# speedup vs baseline: 3.2372x; 1.0056x over previous
import functools

import jax
import jax.numpy as jnp
from jax import lax
from jax.experimental import pallas as pl
from jax.experimental.pallas import tpu as pltpu
from jax.experimental.pallas import tpu_sc as plsc

CHUNK = 128
NBUF = 4
DP = 128


def _make_gather(N, V, D, num_cores, num_subcores):
    NW = num_cores * num_subcores
    per_w = N // NW
    n_chunks = per_w // CHUNK
    assert n_chunks % NBUF == 0
    n_groups = n_chunks // NBUF
    mesh = plsc.VectorSubcoreMesh(core_axis_name="c", subcore_axis_name="s")

    @functools.partial(
        pl.kernel,
        out_type=jax.ShapeDtypeStruct((N, DP), jnp.float32),
        mesh=mesh,
        scratch_types=[
            pltpu.VMEM((per_w,), jnp.int32),
            pltpu.VMEM((NBUF, CHUNK, DP), jnp.float32),
            pltpu.SemaphoreType.DMA((NBUF,)),
            pltpu.SemaphoreType.DMA((NBUF,)),
        ],
    )
    def gather_kernel(idx_hbm, table_hbm, out_hbm, idx_v, rows_v, g_sem, w_sem):
        wid = lax.axis_index("s") * num_cores + lax.axis_index("c")
        base = wid * per_w
        pltpu.sync_copy(idx_hbm.at[pl.ds(base, per_w)], idx_v)

        def start_gather(j, b):
            off = pl.multiple_of(j * CHUNK, CHUNK)
            pltpu.make_async_copy(
                table_hbm.at[idx_v.at[pl.ds(off, CHUNK)]],
                rows_v.at[b],
                g_sem.at[b],
            ).start()

        def wait_gather(b):
            pltpu.make_async_copy(
                table_hbm.at[idx_v.at[pl.ds(0, CHUNK)]],
                rows_v.at[b],
                g_sem.at[b],
            ).wait()

        def start_write(j, b):
            off = pl.multiple_of(j * CHUNK, CHUNK)
            pltpu.make_async_copy(
                rows_v.at[b],
                out_hbm.at[pl.ds(base + off, CHUNK)],
                w_sem.at[b],
            ).start()

        def wait_write(b):
            pltpu.make_async_copy(
                rows_v.at[b],
                out_hbm.at[pl.ds(base, CHUNK)],
                w_sem.at[b],
            ).wait()

        for b in range(NBUF):
            start_gather(b, b)

        def body(g, carry):
            for b in range(NBUF):
                j = g * NBUF + b
                wait_gather(b)
                start_write(j, b)

                @pl.when(g < n_groups - 1)
                def _():
                    wait_write(b)
                    start_gather(j + NBUF, b)

            return carry

        lax.fori_loop(0, n_groups, body, 0)
        for b in range(NBUF):
            wait_write(b)

    return gather_kernel


def _make_reformat(B, L, D, BB):
    def body(i_ref, o_ref):
        o_ref[...] = i_ref[:, :D].reshape(BB, L, D)

    return pl.pallas_call(
        body,
        grid=(B // BB,),
        in_specs=[pl.BlockSpec((BB * L, DP), lambda i: (i, 0))],
        out_specs=pl.BlockSpec((BB, L, D), lambda i: (i, 0, 0)),
        out_shape=jax.ShapeDtypeStruct((B, L, D), jnp.float32),
    )


def kernel(x, table):
    B, L = x.shape
    V, D = table.shape
    N = B * L
    info = plsc.get_sparse_core_info()
    gather = _make_gather(N, V, D, info.num_cores, info.num_subcores)
    table_p = jnp.pad(table, ((0, 0), (0, DP - D)))
    out = gather(x.reshape(N).astype(jnp.int32), table_p)
    return _make_reformat(B, L, D, 128)(out)


# speedup vs baseline: 4.7392x; 1.4640x over previous
import functools

import jax
import jax.numpy as jnp
from jax import lax
from jax.experimental import pallas as pl
from jax.experimental.pallas import tpu as pltpu
from jax.experimental.pallas import tpu_sc as plsc

CHUNK = 128
NBUF = 4
DP = 128


def _make_gather(N, V, D, num_cores, num_subcores):
    NW = num_cores * num_subcores
    per_w = N // NW
    n_chunks = per_w // CHUNK
    assert n_chunks % NBUF == 0
    n_groups = n_chunks // NBUF
    mesh = plsc.VectorSubcoreMesh(core_axis_name="c", subcore_axis_name="s")

    @functools.partial(
        pl.kernel,
        out_type=jax.ShapeDtypeStruct((N, DP), jnp.float32),
        mesh=mesh,
        scratch_types=[
            pltpu.VMEM((per_w,), jnp.int32),
            pltpu.VMEM((NBUF, CHUNK, DP), jnp.float32),
            pltpu.SemaphoreType.DMA((NBUF,)),
            pltpu.SemaphoreType.DMA((NBUF,)),
        ],
    )
    def gather_kernel(idx_hbm, table_hbm, out_hbm, idx_v, rows_v, g_sem, w_sem):
        wid = lax.axis_index("s") * num_cores + lax.axis_index("c")
        base = wid * per_w
        pltpu.sync_copy(idx_hbm.at[pl.ds(base, per_w)], idx_v)

        def start_gather(j, b):
            off = pl.multiple_of(j * CHUNK, CHUNK)
            pltpu.make_async_copy(
                table_hbm.at[idx_v.at[pl.ds(off, CHUNK)]],
                rows_v.at[b],
                g_sem.at[b],
            ).start()

        def wait_gather(b):
            pltpu.make_async_copy(
                table_hbm.at[idx_v.at[pl.ds(0, CHUNK)]],
                rows_v.at[b],
                g_sem.at[b],
            ).wait()

        def start_write(j, b):
            off = pl.multiple_of(j * CHUNK, CHUNK)
            pltpu.make_async_copy(
                rows_v.at[b],
                out_hbm.at[pl.ds(base + off, CHUNK)],
                w_sem.at[b],
            ).start()

        def wait_write(b):
            pltpu.make_async_copy(
                rows_v.at[b],
                out_hbm.at[pl.ds(base, CHUNK)],
                w_sem.at[b],
            ).wait()

        for b in range(NBUF):
            start_gather(b, b)

        def body(g, carry):
            for b in range(NBUF):
                j = g * NBUF + b
                wait_gather(b)
                start_write(j, b)

                @pl.when(g < n_groups - 1)
                def _():
                    wait_write(b)
                    start_gather(j + NBUF, b)

            return carry

        lax.fori_loop(0, n_groups, body, 0)
        for b in range(NBUF):
            wait_write(b)

    return gather_kernel


def _make_reformat(B, L, D, BB):
    def body(i_ref, o_ref):
        o_ref[...] = jnp.transpose(i_ref[...])[None, :D, :]

    return pl.pallas_call(
        body,
        grid=(L, B // BB),
        in_specs=[pl.BlockSpec((BB, DP), lambda l, j: (l * (B // BB) + j, 0))],
        out_specs=pl.BlockSpec((1, D, BB), lambda l, j: (l, 0, j)),
        out_shape=jax.ShapeDtypeStruct((L, D, B), jnp.float32),
    )


def kernel(x, table):
    B, L = x.shape
    V, D = table.shape
    N = B * L
    info = plsc.get_sparse_core_info()
    gather = _make_gather(N, V, D, info.num_cores, info.num_subcores)
    table_p = jnp.pad(table, ((0, 0), (0, DP - D)))
    out = gather(jnp.transpose(x).reshape(N).astype(jnp.int32), table_p)
    out_t = _make_reformat(B, L, D, 4096)(out)
    return jnp.transpose(out_t, (2, 0, 1))


# speedup vs baseline: 5.8495x; 1.2343x over previous
import functools

import jax
import jax.numpy as jnp
from jax import lax
from jax.experimental import pallas as pl
from jax.experimental.pallas import tpu as pltpu
from jax.experimental.pallas import tpu_sc as plsc

CHUNK = 128
NBUF = 4
DP = 128


def _make_gather(N, V, D, num_cores, num_subcores):
    NW = num_cores * num_subcores
    per_w = N // NW
    n_chunks = per_w // CHUNK
    assert n_chunks % NBUF == 0
    n_groups = n_chunks // NBUF
    mesh = plsc.VectorSubcoreMesh(core_axis_name="c", subcore_axis_name="s")

    @functools.partial(
        pl.kernel,
        out_type=jax.ShapeDtypeStruct((N, DP), jnp.float32),
        mesh=mesh,
        scratch_types=[
            pltpu.VMEM((per_w,), jnp.int32),
            pltpu.VMEM((NBUF, CHUNK, DP), jnp.float32),
            pltpu.SemaphoreType.DMA((NBUF,)),
            pltpu.SemaphoreType.DMA((NBUF,)),
        ],
    )
    def gather_kernel(idx_hbm, table_hbm, out_hbm, idx_v, rows_v, g_sem, w_sem):
        wid = lax.axis_index("s") * num_cores + lax.axis_index("c")
        base = wid * per_w
        pltpu.sync_copy(idx_hbm.at[pl.ds(base, per_w)], idx_v)

        def start_gather(j, b):
            off = pl.multiple_of(j * CHUNK, CHUNK)
            pltpu.make_async_copy(
                table_hbm.at[idx_v.at[pl.ds(off, CHUNK)]],
                rows_v.at[b],
                g_sem.at[b],
            ).start()

        def wait_gather(b):
            pltpu.make_async_copy(
                table_hbm.at[idx_v.at[pl.ds(0, CHUNK)]],
                rows_v.at[b],
                g_sem.at[b],
            ).wait()

        def start_write(j, b):
            off = pl.multiple_of(j * CHUNK, CHUNK)
            pltpu.make_async_copy(
                rows_v.at[b],
                out_hbm.at[pl.ds(base + off, CHUNK)],
                w_sem.at[b],
            ).start()

        def wait_write(b):
            pltpu.make_async_copy(
                rows_v.at[b],
                out_hbm.at[pl.ds(base, CHUNK)],
                w_sem.at[b],
            ).wait()

        for b in range(NBUF):
            start_gather(b, b)

        def body(g, carry):
            for b in range(NBUF):
                j = g * NBUF + b
                wait_gather(b)
                start_write(j, b)

                @pl.when(g < n_groups - 1)
                def _():
                    wait_write(b)
                    start_gather(j + NBUF, b)

            return carry

        lax.fori_loop(0, n_groups, body, 0)
        for b in range(NBUF):
            wait_write(b)

    return gather_kernel


def _make_table_prep(V, D, VB):
    def body(i_ref, o_ref):
        o_ref[:, :D] = jnp.transpose(i_ref[...])

    return pl.pallas_call(
        body,
        grid=(pl.cdiv(V, VB),),
        in_specs=[pl.BlockSpec((D, VB), lambda j: (0, j))],
        out_specs=pl.BlockSpec((VB, DP), lambda j: (j, 0)),
        out_shape=jax.ShapeDtypeStruct((V, DP), jnp.float32),
    )


def _make_reformat(B, L, D, BB):
    def body(i_ref, o_ref):
        o_ref[...] = jnp.transpose(i_ref[...])[None, :D, :]

    return pl.pallas_call(
        body,
        grid=(L, B // BB),
        in_specs=[pl.BlockSpec((BB, DP), lambda l, j: (l * (B // BB) + j, 0))],
        out_specs=pl.BlockSpec((1, D, BB), lambda l, j: (l, 0, j)),
        out_shape=jax.ShapeDtypeStruct((L, D, B), jnp.float32),
    )


def kernel(x, table):
    B, L = x.shape
    V, D = table.shape
    N = B * L
    info = plsc.get_sparse_core_info()
    gather = _make_gather(N, V, D, info.num_cores, info.num_subcores)
    table_p = _make_table_prep(V, D, 3200)(jnp.transpose(table))
    out = gather(jnp.transpose(x).reshape(N).astype(jnp.int32), table_p)
    out_t = _make_reformat(B, L, D, 4096)(out)
    return jnp.transpose(out_t, (2, 0, 1))


# speedup vs baseline: 6.2284x; 1.0648x over previous
import functools

import jax
import jax.numpy as jnp
from jax import lax
from jax.experimental import pallas as pl
from jax.experimental.pallas import tpu as pltpu
from jax.experimental.pallas import tpu_sc as plsc

CHUNK = 128
DP = 128
PHASES = 4


def _make_gather(N, V, D, num_cores, num_subcores):
    NW = num_cores * num_subcores
    per_w = N // NW
    n_chunks = per_w // CHUNK
    NBUF = max(n for n in range(2, 7) if n_chunks % n == 0)
    n_groups = n_chunks // NBUF
    mesh = plsc.VectorSubcoreMesh(core_axis_name="c", subcore_axis_name="s")

    @functools.partial(
        pl.kernel,
        out_type=jax.ShapeDtypeStruct((N, DP), jnp.float32),
        mesh=mesh,
        scratch_types=[
            pltpu.VMEM((per_w,), jnp.int32),
            pltpu.VMEM((NBUF, CHUNK, DP), jnp.float32),
            pltpu.SemaphoreType.DMA((NBUF,)),
            pltpu.SemaphoreType.DMA((NBUF,)),
        ],
    )
    def gather_kernel(idx_hbm, table_hbm, out_hbm, idx_v, rows_v, g_sem, w_sem):
        wid = lax.axis_index("s") * num_cores + lax.axis_index("c")
        base = wid * per_w
        pltpu.sync_copy(idx_hbm.at[pl.ds(base, per_w)], idx_v)

        def start_gather(j, b):
            off = pl.multiple_of(j * CHUNK, CHUNK)
            pltpu.make_async_copy(
                table_hbm.at[idx_v.at[pl.ds(off, CHUNK)]],
                rows_v.at[b],
                g_sem.at[b],
            ).start()

        def wait_gather(b):
            pltpu.make_async_copy(
                table_hbm.at[idx_v.at[pl.ds(0, CHUNK)]],
                rows_v.at[b],
                g_sem.at[b],
            ).wait()

        def start_write(j, b):
            off = pl.multiple_of(j * CHUNK, CHUNK)
            pltpu.make_async_copy(
                rows_v.at[b],
                out_hbm.at[pl.ds(base + off, CHUNK)],
                w_sem.at[b],
            ).start()

        def wait_write(b):
            pltpu.make_async_copy(
                rows_v.at[b],
                out_hbm.at[pl.ds(base, CHUNK)],
                w_sem.at[b],
            ).wait()

        for b in range(NBUF):
            start_gather(b, b)

        def body(g, carry):
            for b in range(NBUF):
                j = g * NBUF + b
                wait_gather(b)
                start_write(j, b)

                @pl.when(g < n_groups - 1)
                def _():
                    wait_write(b)
                    start_gather(j + NBUF, b)

            return carry

        lax.fori_loop(0, n_groups, body, 0)
        for b in range(NBUF):
            wait_write(b)

    return gather_kernel


def _make_table_prep(V, D, VB):
    def body(i_ref, o_ref):
        o_ref[:, :D] = jnp.transpose(i_ref[...])

    return pl.pallas_call(
        body,
        grid=(pl.cdiv(V, VB),),
        in_specs=[pl.BlockSpec((D, VB), lambda j: (0, j))],
        out_specs=pl.BlockSpec((VB, DP), lambda j: (j, 0)),
        out_shape=jax.ShapeDtypeStruct((V, DP), jnp.float32),
    )


def _make_reformat(B, L, D, Bp, p, BB, aliased):
    boff = (p * Bp) // BB

    def body(*refs):
        i_ref, o_ref = refs[0], refs[-1]
        o_ref[...] = jnp.transpose(i_ref[...])[None, :D, :]

    in_specs = [pl.BlockSpec((BB, DP), lambda l, j: (l * (Bp // BB) + j, 0))]
    nargs = 1
    if aliased:
        in_specs.append(pl.BlockSpec(memory_space=pl.ANY))
        nargs = 2
    return pl.pallas_call(
        body,
        grid=(L, Bp // BB),
        in_specs=in_specs,
        out_specs=pl.BlockSpec((1, D, BB), lambda l, j: (l, 0, boff + j)),
        out_shape=jax.ShapeDtypeStruct((L, D, B), jnp.float32),
        input_output_aliases={1: 0} if aliased else {},
    )


def kernel(x, table):
    B, L = x.shape
    V, D = table.shape
    N = B * L
    Bp = B // PHASES
    info = plsc.get_sparse_core_info()
    gather = _make_gather(L * Bp, V, D, info.num_cores, info.num_subcores)
    table_p = _make_table_prep(V, D, 3200)(jnp.transpose(table))
    xt = jnp.transpose(x).astype(jnp.int32)
    out_t = None
    for p in range(PHASES):
        idx_p = xt[:, p * Bp:(p + 1) * Bp].reshape(L * Bp)
        g_p = gather(idx_p, table_p)
        reformat = _make_reformat(B, L, D, Bp, p, 4096, aliased=p > 0)
        out_t = reformat(g_p) if p == 0 else reformat(g_p, out_t)
    return jnp.transpose(out_t, (2, 0, 1))
